# baseline (speedup 1.0000x reference)
import functools

import jax
import jax.numpy as jnp
from jax import lax
from jax.experimental import pallas as pl
from jax.experimental.pallas import tpu as pltpu
from jax.experimental.pallas import tpu_sc as plsc

B, S, D = 4, 2048, 1024
NC, NS = 2, 16
NW = NC * NS
ROWS_PER_W = S // NW
CH = 8
CHD = CH * D
NK = ROWS_PER_W // CH
NJOBS = B * NK
NBUF = 4
LA = 2

_mesh = plsc.VectorSubcoreMesh(
    core_axis_name="c", subcore_axis_name="s", num_cores=NC, num_subcores=NS
)


def _sc_body(in_hbm, pos_hbm, out_hbm, pos_v, *rest):
    bufs = rest[:NBUF]
    in_sems = rest[NBUF : 2 * NBUF]
    out_sems = rest[2 * NBUF : 3 * NBUF]
    wid = lax.axis_index("s") * NC + lax.axis_index("c")
    base = wid * ROWS_PER_W

    pltpu.sync_copy(pos_hbm.at[pl.ds(base * D, ROWS_PER_W * D)], pos_v)

    def job_off(j):
        b, k = divmod(j, NK)
        return (b * S + k * CH) * D + base * D

    def start_in(j, t):
        pltpu.async_copy(in_hbm.at[pl.ds(job_off(j), CHD)], bufs[t], in_sems[t])

    def wait_in(t):
        pltpu.make_async_copy(
            in_hbm.at[pl.ds(0, CHD)], bufs[t], in_sems[t]
        ).wait()

    def start_out(j, t):
        pltpu.async_copy(
            bufs[t], out_hbm.at[pl.ds(job_off(j), CHD)], out_sems[t]
        )

    def wait_out(t):
        pltpu.make_async_copy(
            in_hbm.at[pl.ds(0, CHD)], bufs[t], out_sems[t]
        ).wait()

    for j in range(LA):
        start_in(j, j % NBUF)
    for j in range(NJOBS):
        t = j % NBUF
        jn = j + LA
        if jn < NJOBS:
            tn = jn % NBUF
            if jn - NBUF >= 0:
                wait_out(tn)
            start_in(jn, tn)
        wait_in(t)
        p0 = (j % NK) * CHD
        buf = bufs[t]

        @plsc.parallel_loop(0, CHD, 16, unroll=4)
        def _(i):
            plsc.addupdate(buf.at[pl.ds(i, 16)], pos_v[pl.ds(p0 + i, 16)])

        start_out(j, t)
    for j in range(NJOBS - NBUF, NJOBS):
        wait_out(j % NBUF)


def _sc_add(inputs, pos_table):
    f = pl.kernel(
        _sc_body,
        out_type=jax.ShapeDtypeStruct((B * S * D,), jnp.float32),
        mesh=_mesh,
        scratch_types=[pltpu.VMEM((ROWS_PER_W * D,), jnp.float32)]
        + [pltpu.VMEM((CHD,), jnp.float32) for _ in range(NBUF)]
        + [pltpu.SemaphoreType.DMA for _ in range(2 * NBUF)],
    )
    out = f(inputs.reshape(-1), pos_table.reshape(-1))
    return out.reshape(B, S, D)


def _tc_add_kernel(x_ref, p_ref, o_ref):
    o_ref[...] = x_ref[...] + p_ref[...]


def _tc_add(inputs, pos_table):
    b, l, d = inputs.shape
    bl = 2048
    grid = (l // bl, b)
    return pl.pallas_call(
        _tc_add_kernel,
        grid=grid,
        in_specs=[
            pl.BlockSpec((1, bl, d), lambda i, bb: (bb, i, 0)),
            pl.BlockSpec((bl, d), lambda i, bb: (i, 0)),
        ],
        out_specs=pl.BlockSpec((1, bl, d), lambda i, bb: (bb, i, 0)),
        out_shape=jax.ShapeDtypeStruct(inputs.shape, inputs.dtype),
    )(inputs, pos_table)


def kernel(inputs, pos_table):
    return _sc_add(inputs, pos_table)
